# baseline (speedup 1.0000x reference)
import functools

import jax
import jax.numpy as jnp
from jax import lax
from jax.experimental import pallas as pl
from jax.experimental.pallas import tpu as pltpu
from jax.experimental.pallas import tpu_sc as plsc

N = 10000
E = 640000
K = 16
HID = 64
OUT = 64
IN = 3

NC = 2
NS = 16
NW = NC * NS
L = 16
NPW = 313
NP = NW * NPW


def _take16(x, idx):
    return x.at[idx].get(mode="promise_in_bounds")


def _cumsum16(x):
    iota = lax.iota(jnp.int32, L)
    for s in (1, 2, 4, 8):
        sh = _take16(x, jnp.maximum(iota - s, 0))
        x = x + jnp.where(iota >= s, sh, 0)
    return x


def _splat16(x, i):
    return _take16(x, jnp.full((L,), i, jnp.int32))


def _scalar16(x):
    return jnp.squeeze(lax.slice(x, (0,), (1,)))


_SEG_CH = 8000
_SEG_CAP = 24576
_XROWS = 320


def _segmax_body(dst_hbm, h_hbm, out_hbm, dstbuf, packed, x1l, rows, sems):
    wid = lax.axis_index("s") * NC + lax.axis_index("c")
    base = wid * NPW

    def _z(i, _):
        x1l[pl.ds(i * L, L)] = jnp.zeros((L,), jnp.float32)
        return 0
    lax.fori_loop(0, (_XROWS * HID) // L, _z, 0)

    nchunk = E // _SEG_CH
    cp = pltpu.async_copy(dst_hbm.at[pl.ds(0, _SEG_CH)],
                          dstbuf.at[pl.ds(0, _SEG_CH)], sems.at[0])

    def _chunk(c, cur):
        buf = lax.rem(c, 2)
        nbuf = lax.rem(c + 1, 2)
        pltpu.make_async_copy(dst_hbm.at[pl.ds(c * _SEG_CH, _SEG_CH)],
                              dstbuf.at[pl.ds(buf * _SEG_CH, _SEG_CH)],
                              sems.at[0]).wait()

        @pl.when(c + 1 < nchunk)
        def _():
            pltpu.async_copy(dst_hbm.at[pl.ds((c + 1) * _SEG_CH, _SEG_CH)],
                             dstbuf.at[pl.ds(nbuf * _SEG_CH, _SEG_CH)],
                             sems.at[0])

        def _step(s, cur):
            v = dstbuf[pl.ds(buf * _SEG_CH + s * L, L)]
            l = v - base
            m = (l >= 0) & (l < NPW)
            e = c * _SEG_CH + s * L + lax.iota(jnp.int32, L)
            pk = (e << 9) | jnp.where(m, l, 0)
            mi = jnp.where(m, 1, 0)
            incl = _cumsum16(mi)
            pos = jnp.where(m, cur + incl - mi,
                            _SEG_CAP - L + lax.iota(jnp.int32, L))
            plsc.store_scatter(packed, [pos], pk)
            return cur + _splat16(incl, 15)

        return lax.fori_loop(0, _SEG_CH // L, _step, cur)

    curv = lax.fori_loop(0, nchunk, _chunk, jnp.zeros((L,), jnp.int32))

    plsc.store_scatter(packed, [curv + lax.iota(jnp.int32, L)],
                       jnp.full((L,), 314, jnp.int32))
    cur = _scalar16(curv)
    ngrp = (cur + L - 1) >> 4

    col = lax.iota(jnp.int32, L)

    def _grp(g, _):
        pv = packed[pl.ds(g * L, L)]
        ev = pv >> 9
        lv = pv & 511
        pltpu.async_copy(h_hbm.at[ev], rows, sems.at[1]).wait()
        for i in range(L):
            lsp = _splat16(lv, i)
            for ch in range(HID // L):
                idx = lsp * HID + ch * L + col
                curv = plsc.load_gather(x1l, [idx])
                hv = rows[i, pl.ds(ch * L, L)]
                plsc.store_scatter(x1l, [idx], jnp.maximum(curv, hv))
        return 0

    lax.fori_loop(0, ngrp, _grp, 0)

    pltpu.sync_copy(x1l.at[pl.ds(0, NPW * HID)],
                    out_hbm.at[pl.ds(base * HID, NPW * HID)])


def _segmax(dst, h):
    mesh = plsc.VectorSubcoreMesh(core_axis_name="c", subcore_axis_name="s")
    kfn = pl.kernel(
        _segmax_body,
        out_type=jax.ShapeDtypeStruct((NP * HID,), jnp.float32),
        mesh=mesh,
        compiler_params=pltpu.CompilerParams(
            needs_layout_passes=False, use_tc_tiling_on_sc=False),
        scratch_types=[
            pltpu.VMEM((2 * _SEG_CH,), jnp.int32),
            pltpu.VMEM((_SEG_CAP,), jnp.int32),
            pltpu.VMEM((_XROWS * HID,), jnp.float32),
            pltpu.VMEM((L, HID), jnp.float32),
            pltpu.SemaphoreType.DMA((2,)),
        ],
    )
    return kfn(dst, h)


def _mlp2_body(m_ref, wa_ref, ba_ref, wb_ref, bb_ref, o_ref):
    m = m_ref[...]
    h = jnp.maximum(jnp.dot(m, wa_ref[...], preferred_element_type=jnp.float32)
                    + ba_ref[...], 0.0)
    h = jnp.maximum(jnp.dot(h, wb_ref[...], preferred_element_type=jnp.float32)
                    + bb_ref[...], 0.0)
    o_ref[...] = h


def _mlp2(m, wa, ba, wb, bb, block):
    mm, d = m.shape
    dmid = wa.shape[1]
    dout = wb.shape[1]
    return pl.pallas_call(
        _mlp2_body,
        grid=(mm // block,),
        in_specs=[
            pl.BlockSpec((block, d), lambda i: (i, 0)),
            pl.BlockSpec((d, dmid), lambda i: (0, 0)),
            pl.BlockSpec((1, dmid), lambda i: (0, 0)),
            pl.BlockSpec((dmid, dout), lambda i: (0, 0)),
            pl.BlockSpec((1, dout), lambda i: (0, 0)),
        ],
        out_specs=pl.BlockSpec((block, dout), lambda i: (i, 0)),
        out_shape=jax.ShapeDtypeStruct((mm, dout), jnp.float32),
    )(m, wa, ba.reshape(1, dmid), wb, bb.reshape(1, dout))


def _mlp2_max_body(m_ref, wa_ref, ba_ref, wb_ref, bb_ref, o_ref, *, bn):
    m = m_ref[...]
    h = jnp.maximum(jnp.dot(m, wa_ref[...], preferred_element_type=jnp.float32)
                    + ba_ref[...], 0.0)
    h = jnp.maximum(jnp.dot(h, wb_ref[...], preferred_element_type=jnp.float32)
                    + bb_ref[...], 0.0)
    o_ref[...] = jnp.max(h.reshape(bn, K, -1), axis=1)


def _mlp2_max(m, wa, ba, wb, bb, bn):
    n = m.shape[0] // K
    d = m.shape[1]
    dmid = wa.shape[1]
    dout = wb.shape[1]
    return pl.pallas_call(
        functools.partial(_mlp2_max_body, bn=bn),
        grid=(n // bn,),
        in_specs=[
            pl.BlockSpec((bn * K, d), lambda i: (i, 0)),
            pl.BlockSpec((d, dmid), lambda i: (0, 0)),
            pl.BlockSpec((1, dmid), lambda i: (0, 0)),
            pl.BlockSpec((dmid, dout), lambda i: (0, 0)),
            pl.BlockSpec((1, dout), lambda i: (0, 0)),
        ],
        out_specs=pl.BlockSpec((bn, dout), lambda i: (i, 0)),
        out_shape=jax.ShapeDtypeStruct((n, dout), jnp.float32),
    )(m, wa, ba.reshape(1, dmid), wb, bb.reshape(1, dout))


def kernel(x, edge_index, W1, b1, W2, b2, W3, b3, W4, b4):
    src = edge_index[0]
    dst = edge_index[1]

    xi = x[dst]
    xj = x[src]
    msg = jnp.concatenate([xi, xj - xi], axis=-1)
    h = _mlp2(msg, W1, b1, W2, b2, block=2560)
    x1 = _segmax(dst, h).reshape(NP, HID)[:N]

    sq = jnp.sum(x1 * x1, axis=1)
    def body(q):
        d2 = jnp.sum(q * q, axis=1)[:, None] - 2.0 * (q @ x1.T) + sq[None, :]
        _, idx = jax.lax.top_k(-d2, K)
        return idx
    qs = x1.reshape(N // 1000, 1000, HID)
    idx = jax.lax.map(body, qs).reshape(N, K)

    nj = x1[idx]
    ni = jnp.broadcast_to(x1[:, None, :], nj.shape)
    msg2 = jnp.concatenate([ni, nj - ni], axis=-1).reshape(N * K, 2 * HID)
    out = _mlp2_max(msg2, W3, b3, W4, b4, bn=80)
    return out


# speedup vs baseline: 1.1131x; 1.1131x over previous
import functools

import jax
import jax.numpy as jnp
from jax import lax
from jax.experimental import pallas as pl
from jax.experimental.pallas import tpu as pltpu
from jax.experimental.pallas import tpu_sc as plsc

N = 10000
E = 640000
K = 16
HID = 64
OUT = 64
IN = 3

NC = 2
NS = 16
NW = NC * NS
L = 16
NPW = 313
NP = NW * NPW


def _take16(x, idx):
    return x.at[idx].get(mode="promise_in_bounds")


def _cumsum16(x):
    iota = lax.iota(jnp.int32, L)
    for s in (1, 2, 4, 8):
        sh = _take16(x, jnp.maximum(iota - s, 0))
        x = x + jnp.where(iota >= s, sh, 0)
    return x


def _splat16(x, i):
    return _take16(x, jnp.full((L,), i, jnp.int32))


def _scalar16(x):
    return jnp.squeeze(lax.slice(x, (0,), (1,)))


_SEG_CH = 8000
_SEG_CAP = 24576
_XROWS = 320


def _segmax_body(dst_hbm, h_hbm, out_hbm, dstbuf, packed, x1l, rows, sems):
    wid = lax.axis_index("s") * NC + lax.axis_index("c")
    base = wid * NPW

    def _z(i, _):
        x1l[pl.ds(i * L, L)] = jnp.zeros((L,), jnp.float32)
        return 0
    lax.fori_loop(0, (_XROWS * HID) // L, _z, 0)

    nchunk = E // _SEG_CH
    cp = pltpu.async_copy(dst_hbm.at[pl.ds(0, _SEG_CH)],
                          dstbuf.at[pl.ds(0, _SEG_CH)], sems.at[0])

    def _chunk(c, cur):
        buf = lax.rem(c, 2)
        nbuf = lax.rem(c + 1, 2)
        pltpu.make_async_copy(dst_hbm.at[pl.ds(c * _SEG_CH, _SEG_CH)],
                              dstbuf.at[pl.ds(buf * _SEG_CH, _SEG_CH)],
                              sems.at[0]).wait()

        @pl.when(c + 1 < nchunk)
        def _():
            pltpu.async_copy(dst_hbm.at[pl.ds((c + 1) * _SEG_CH, _SEG_CH)],
                             dstbuf.at[pl.ds(nbuf * _SEG_CH, _SEG_CH)],
                             sems.at[0])

        def _step(s, cur):
            v = dstbuf[pl.ds(buf * _SEG_CH + s * L, L)]
            l = v - base
            m = (l >= 0) & (l < NPW)
            e = c * _SEG_CH + s * L + lax.iota(jnp.int32, L)
            pk = (e << 9) | jnp.where(m, l, 0)
            mi = jnp.where(m, 1, 0)
            incl = _cumsum16(mi)
            pos = jnp.where(m, cur + incl - mi,
                            _SEG_CAP - L + lax.iota(jnp.int32, L))
            plsc.store_scatter(packed, [pos], pk)
            return cur + _splat16(incl, 15)

        return lax.fori_loop(0, _SEG_CH // L, _step, cur)

    curv = lax.fori_loop(0, nchunk, _chunk, jnp.zeros((L,), jnp.int32))

    plsc.store_scatter(packed, [curv + lax.iota(jnp.int32, L)],
                       jnp.full((L,), 314, jnp.int32))
    cur = _scalar16(curv)
    ngrp = (cur + L - 1) >> 4

    col = lax.iota(jnp.int32, L)

    def _grp(g, _):
        pv = packed[pl.ds(g * L, L)]
        ev = pv >> 9
        lv = pv & 511
        pltpu.async_copy(h_hbm.at[ev], rows, sems.at[1]).wait()
        for i in range(L):
            lsp = _splat16(lv, i)
            for ch in range(HID // L):
                idx = lsp * HID + ch * L + col
                curv = plsc.load_gather(x1l, [idx])
                hv = rows[i, pl.ds(ch * L, L)]
                plsc.store_scatter(x1l, [idx], jnp.maximum(curv, hv))
        return 0

    lax.fori_loop(0, ngrp, _grp, 0)

    pltpu.sync_copy(x1l.at[pl.ds(0, NPW * HID)],
                    out_hbm.at[pl.ds(base * HID, NPW * HID)])


def _segmax(dst, h):
    mesh = plsc.VectorSubcoreMesh(core_axis_name="c", subcore_axis_name="s")
    kfn = pl.kernel(
        _segmax_body,
        out_type=jax.ShapeDtypeStruct((NP * HID,), jnp.float32),
        mesh=mesh,
        compiler_params=pltpu.CompilerParams(
            needs_layout_passes=False, use_tc_tiling_on_sc=False),
        scratch_types=[
            pltpu.VMEM((2 * _SEG_CH,), jnp.int32),
            pltpu.VMEM((_SEG_CAP,), jnp.int32),
            pltpu.VMEM((_XROWS * HID,), jnp.float32),
            pltpu.VMEM((L, HID), jnp.float32),
            pltpu.SemaphoreType.DMA((2,)),
        ],
    )
    return kfn(dst, h)


def _mlp2_body(m_ref, wa_ref, ba_ref, wb_ref, bb_ref, o_ref):
    m = m_ref[...]
    h = jnp.maximum(jnp.dot(m, wa_ref[...], preferred_element_type=jnp.float32)
                    + ba_ref[...], 0.0)
    h = jnp.maximum(jnp.dot(h, wb_ref[...], preferred_element_type=jnp.float32)
                    + bb_ref[...], 0.0)
    o_ref[...] = h


def _mlp2(m, wa, ba, wb, bb, block):
    mm, d = m.shape
    dmid = wa.shape[1]
    dout = wb.shape[1]
    return pl.pallas_call(
        _mlp2_body,
        grid=(mm // block,),
        in_specs=[
            pl.BlockSpec((block, d), lambda i: (i, 0)),
            pl.BlockSpec((d, dmid), lambda i: (0, 0)),
            pl.BlockSpec((1, dmid), lambda i: (0, 0)),
            pl.BlockSpec((dmid, dout), lambda i: (0, 0)),
            pl.BlockSpec((1, dout), lambda i: (0, 0)),
        ],
        out_specs=pl.BlockSpec((block, dout), lambda i: (i, 0)),
        out_shape=jax.ShapeDtypeStruct((mm, dout), jnp.float32),
    )(m, wa, ba.reshape(1, dmid), wb, bb.reshape(1, dout))


def _mlp2_max_body(m_ref, wa_ref, ba_ref, wb_ref, bb_ref, o_ref, *, bn):
    m = m_ref[...]
    h = jnp.maximum(jnp.dot(m, wa_ref[...], preferred_element_type=jnp.float32)
                    + ba_ref[...], 0.0)
    h = jnp.maximum(jnp.dot(h, wb_ref[...], preferred_element_type=jnp.float32)
                    + bb_ref[...], 0.0)
    o_ref[...] = jnp.max(h.reshape(bn, K, -1), axis=1)


def _mlp2_max(m, wa, ba, wb, bb, bn):
    n = m.shape[0] // K
    d = m.shape[1]
    dmid = wa.shape[1]
    dout = wb.shape[1]
    return pl.pallas_call(
        functools.partial(_mlp2_max_body, bn=bn),
        grid=(n // bn,),
        in_specs=[
            pl.BlockSpec((bn * K, d), lambda i: (i, 0)),
            pl.BlockSpec((d, dmid), lambda i: (0, 0)),
            pl.BlockSpec((1, dmid), lambda i: (0, 0)),
            pl.BlockSpec((dmid, dout), lambda i: (0, 0)),
            pl.BlockSpec((1, dout), lambda i: (0, 0)),
        ],
        out_specs=pl.BlockSpec((bn, dout), lambda i: (i, 0)),
        out_shape=jax.ShapeDtypeStruct((n, dout), jnp.float32),
    )(m, wa, ba.reshape(1, dmid), wb, bb.reshape(1, dout))


def _d2_body(q_ref, x1_ref, o_ref):
    q = q_ref[...]
    x1 = x1_ref[...]
    qq = jnp.sum(q * q, axis=1)
    sq = jnp.sum(x1 * x1, axis=1)
    dot = lax.dot_general(q, x1, (((1,), (1,)), ((), ())),
                          preferred_element_type=jnp.float32)
    o_ref[...] = qq[:, None] - 2.0 * dot + sq[None, :]


def _d2(x1, bq):
    return pl.pallas_call(
        _d2_body,
        grid=(N // bq,),
        in_specs=[
            pl.BlockSpec((bq, HID), lambda i: (i, 0)),
            pl.BlockSpec((N, HID), lambda i: (0, 0)),
        ],
        out_specs=pl.BlockSpec((bq, N), lambda i: (i, 0)),
        out_shape=jax.ShapeDtypeStruct((N, N), jnp.float32),
    )(x1, x1)


_ROWPAD = 2 * N
_SCAP = 2048


def _argmin16(bv, biv):
    iota = lax.iota(jnp.int32, L)
    for s in (1, 2, 4, 8):
        ov = _take16(bv, iota ^ s)
        oi = _take16(biv, iota ^ s)
        take = (ov < bv) | ((ov == bv) & (oi < biv))
        bv = jnp.where(take, ov, bv)
        biv = jnp.where(take, oi, biv)
    return bv, biv


def _topk_body(d2_hbm, x1m_hbm, out_hbm,
               rowbuf, svbuf, nbr, nib, m2b, sems):
    wid = lax.axis_index("s") * NC + lax.axis_index("c")
    base = wid * NPW
    nq = jnp.minimum(NPW, N - base)
    iota = lax.iota(jnp.int32, L)
    inf = jnp.full((L,), jnp.inf, jnp.float32)
    rowbuf[pl.ds(_ROWPAD, L)] = inf

    cp = pltpu.async_copy(d2_hbm.at[pl.ds(base * N, N)],
                          rowbuf.at[pl.ds(0, N)], sems.at[0])

    def _row(q, _):
        buf = lax.rem(q, 2)
        bufbase = buf * N
        pltpu.make_async_copy(d2_hbm.at[pl.ds((base + q) * N, N)],
                              rowbuf.at[pl.ds(bufbase, N)], sems.at[0]).wait()

        @pl.when(q + 1 < nq)
        def _():
            pltpu.async_copy(d2_hbm.at[pl.ds((base + q + 1) * N, N)],
                             rowbuf.at[pl.ds(lax.rem(q + 1, 2) * N, N)],
                             sems.at[0])

        def _p1(s, mv):
            v = rowbuf[pl.ds(bufbase + s * L, L)]
            m = v < mv
            return jnp.where(m, v, mv)

        mv = lax.fori_loop(0, N // L, _p1, inf)
        thr = mv
        for s in (1, 2, 4, 8):
            thr = jnp.maximum(thr, _take16(thr, iota ^ s))

        def _p2(s, cur):
            v = rowbuf[pl.ds(bufbase + s * L, L)]
            m = v <= thr

            def _ins():
                mi = jnp.where(m, 1, 0)
                incl = _cumsum16(mi)
                pos = jnp.where(m, cur + incl - mi, _SCAP - L + iota)
                plsc.store_scatter(svbuf, [pos], bufbase + s * L + iota)
                return cur + _splat16(incl, 15)

            return lax.cond(jnp.any(m), _ins, lambda: cur)

        curv = lax.fori_loop(0, N // L, _p2, jnp.zeros((L,), jnp.int32))
        plsc.store_scatter(svbuf, [curv + iota],
                           jnp.full((L,), _ROWPAD, jnp.int32))
        ns = (_scalar16(curv) + L - 1) >> 4

        def _sel(t, res):
            def _scan(g, c):
                bv, biv = c
                iv = svbuf[pl.ds(g * L, L)]
                dv = plsc.load_gather(rowbuf, [iv])
                take = (dv < bv) | ((dv == bv) & (iv < biv))
                return jnp.where(take, dv, bv), jnp.where(take, iv, biv)

            bv, biv = lax.fori_loop(0, ns, _scan, (inf, jnp.full((L,), _ROWPAD, jnp.int32)))
            bv, biv = _argmin16(bv, biv)
            plsc.store_scatter(rowbuf, [biv], inf)
            return jnp.where(iota == t, biv, res)

        res = lax.fori_loop(0, K, _sel, jnp.zeros((L,), jnp.int32))
        nbrs = res - bufbase

        pltpu.async_copy(x1m_hbm.at[nbrs], nbr, sems.at[1]).wait()
        pltpu.sync_copy(x1m_hbm.at[pl.ds(base + q, 1)], nib)
        for i in range(K):
            for c in range(HID // L):
                niv = nib[0, pl.ds(c * L, L)]
                njv = nbr[i, pl.ds(c * L, L)]
                m2b[pl.ds(i * 2 * HID + c * L, L)] = niv
                m2b[pl.ds(i * 2 * HID + HID + c * L, L)] = njv - niv
        pltpu.sync_copy(m2b, out_hbm.at[pl.ds((base + q) * K * 2 * HID,
                                              K * 2 * HID)])
        return 0

    lax.fori_loop(0, nq, _row, 0)


def _topk_msg2(d2, x1m):
    mesh = plsc.VectorSubcoreMesh(core_axis_name="c", subcore_axis_name="s")
    kfn = pl.kernel(
        _topk_body,
        out_type=jax.ShapeDtypeStruct((N * K * 2 * HID,), jnp.float32),
        mesh=mesh,
        compiler_params=pltpu.CompilerParams(
            needs_layout_passes=False, use_tc_tiling_on_sc=False),
        scratch_types=[
            pltpu.VMEM((_ROWPAD + L,), jnp.float32),
            pltpu.VMEM((_SCAP,), jnp.int32),
            pltpu.VMEM((K, HID), jnp.float32),
            pltpu.VMEM((1, HID), jnp.float32),
            pltpu.VMEM((K * 2 * HID,), jnp.float32),
            pltpu.SemaphoreType.DMA((2,)),
        ],
    )
    return kfn(d2.reshape(N * N), x1m)


def kernel(x, edge_index, W1, b1, W2, b2, W3, b3, W4, b4):
    src = edge_index[0]
    dst = edge_index[1]

    xi = x[dst]
    xj = x[src]
    msg = jnp.concatenate([xi, xj - xi], axis=-1)
    h = _mlp2(msg, W1, b1, W2, b2, block=2560)
    x1 = _segmax(dst, h).reshape(NP, HID)[:N]

    d2 = _d2(x1, bq=400)
    msg2 = _topk_msg2(d2, x1).reshape(N * K, 2 * HID)
    out = _mlp2_max(msg2, W3, b3, W4, b4, bn=80)
    return out


# speedup vs baseline: 1.8306x; 1.6446x over previous
import functools

import jax
import jax.numpy as jnp
from jax import lax
from jax.experimental import pallas as pl
from jax.experimental.pallas import tpu as pltpu
from jax.experimental.pallas import tpu_sc as plsc

N = 10000
E = 640000
K = 16
HID = 64
OUT = 64
IN = 3

NC = 2
NS = 16
NW = NC * NS
L = 16
NPW = 313
NP = NW * NPW


def _take16(x, idx):
    return x.at[idx].get(mode="promise_in_bounds")


def _cumsum16(x):
    iota = lax.iota(jnp.int32, L)
    for s in (1, 2, 4, 8):
        sh = _take16(x, jnp.maximum(iota - s, 0))
        x = x + jnp.where(iota >= s, sh, 0)
    return x


def _splat16(x, i):
    return _take16(x, jnp.full((L,), i, jnp.int32))


def _scalar16(x):
    return jnp.squeeze(lax.slice(x, (0,), (1,)))


_SEG_CH = 8000
_SEG_CAP = 24576
_XROWS = 320


def _segmax_body(dst_hbm, h_hbm, out_hbm, dstbuf, packed, x1l, rows, sems):
    wid = lax.axis_index("s") * NC + lax.axis_index("c")
    base = wid * NPW

    def _z(i, _):
        x1l[pl.ds(i * L, L)] = jnp.zeros((L,), jnp.float32)
        return 0
    lax.fori_loop(0, (_XROWS * HID) // L, _z, 0)

    nchunk = E // _SEG_CH
    cp = pltpu.async_copy(dst_hbm.at[pl.ds(0, _SEG_CH)],
                          dstbuf.at[pl.ds(0, _SEG_CH)], sems.at[0])

    def _chunk(c, cur):
        buf = lax.rem(c, 2)
        nbuf = lax.rem(c + 1, 2)
        pltpu.make_async_copy(dst_hbm.at[pl.ds(c * _SEG_CH, _SEG_CH)],
                              dstbuf.at[pl.ds(buf * _SEG_CH, _SEG_CH)],
                              sems.at[0]).wait()

        @pl.when(c + 1 < nchunk)
        def _():
            pltpu.async_copy(dst_hbm.at[pl.ds((c + 1) * _SEG_CH, _SEG_CH)],
                             dstbuf.at[pl.ds(nbuf * _SEG_CH, _SEG_CH)],
                             sems.at[0])

        def _step(s, cur):
            v = dstbuf[pl.ds(buf * _SEG_CH + s * L, L)]
            l = v - base
            m = (l >= 0) & (l < NPW)
            e = c * _SEG_CH + s * L + lax.iota(jnp.int32, L)
            pk = (e << 9) | jnp.where(m, l, 0)
            mi = jnp.where(m, 1, 0)
            incl = _cumsum16(mi)
            pos = jnp.where(m, cur + incl - mi,
                            _SEG_CAP - L + lax.iota(jnp.int32, L))
            plsc.store_scatter(packed, [pos], pk)
            return cur + _splat16(incl, 15)

        return lax.fori_loop(0, _SEG_CH // L, _step, cur)

    curv = lax.fori_loop(0, nchunk, _chunk, jnp.zeros((L,), jnp.int32))

    plsc.store_scatter(packed, [curv + lax.iota(jnp.int32, L)],
                       jnp.full((L,), 314, jnp.int32))
    cur = _scalar16(curv)
    ngrp = (cur + L - 1) >> 4

    col = lax.iota(jnp.int32, L)

    def _grp(g, _):
        pv = packed[pl.ds(g * L, L)]
        ev = pv >> 9
        lv = pv & 511
        pltpu.async_copy(h_hbm.at[ev], rows, sems.at[1]).wait()
        for i in range(L):
            lsp = _splat16(lv, i)
            for ch in range(HID // L):
                idx = lsp * HID + ch * L + col
                curv = plsc.load_gather(x1l, [idx])
                hv = rows[i, pl.ds(ch * L, L)]
                plsc.store_scatter(x1l, [idx], jnp.maximum(curv, hv))
        return 0

    lax.fori_loop(0, ngrp, _grp, 0)

    pltpu.sync_copy(x1l.at[pl.ds(0, NPW * HID)],
                    out_hbm.at[pl.ds(base * HID, NPW * HID)])


def _segmax(dst, h):
    mesh = plsc.VectorSubcoreMesh(core_axis_name="c", subcore_axis_name="s")
    kfn = pl.kernel(
        _segmax_body,
        out_type=jax.ShapeDtypeStruct((NP * HID,), jnp.float32),
        mesh=mesh,
        compiler_params=pltpu.CompilerParams(
            needs_layout_passes=False, use_tc_tiling_on_sc=False),
        scratch_types=[
            pltpu.VMEM((2 * _SEG_CH,), jnp.int32),
            pltpu.VMEM((_SEG_CAP,), jnp.int32),
            pltpu.VMEM((_XROWS * HID,), jnp.float32),
            pltpu.VMEM((L, HID), jnp.float32),
            pltpu.SemaphoreType.DMA((2,)),
        ],
    )
    return kfn(dst, h)


def _mlp2_body(m_ref, wa_ref, ba_ref, wb_ref, bb_ref, o_ref):
    m = m_ref[...]
    h = jnp.maximum(jnp.dot(m, wa_ref[...], preferred_element_type=jnp.float32)
                    + ba_ref[...], 0.0)
    h = jnp.maximum(jnp.dot(h, wb_ref[...], preferred_element_type=jnp.float32)
                    + bb_ref[...], 0.0)
    o_ref[...] = h


def _mlp2(m, wa, ba, wb, bb, block):
    mm, d = m.shape
    dmid = wa.shape[1]
    dout = wb.shape[1]
    return pl.pallas_call(
        _mlp2_body,
        grid=(mm // block,),
        in_specs=[
            pl.BlockSpec((block, d), lambda i: (i, 0)),
            pl.BlockSpec((d, dmid), lambda i: (0, 0)),
            pl.BlockSpec((1, dmid), lambda i: (0, 0)),
            pl.BlockSpec((dmid, dout), lambda i: (0, 0)),
            pl.BlockSpec((1, dout), lambda i: (0, 0)),
        ],
        out_specs=pl.BlockSpec((block, dout), lambda i: (i, 0)),
        out_shape=jax.ShapeDtypeStruct((mm, dout), jnp.float32),
    )(m, wa, ba.reshape(1, dmid), wb, bb.reshape(1, dout))


def _mlp2_max_body(m_ref, wa_ref, ba_ref, wb_ref, bb_ref, o_ref, *, bn):
    m = m_ref[...]
    h = jnp.maximum(jnp.dot(m, wa_ref[...], preferred_element_type=jnp.float32)
                    + ba_ref[...], 0.0)
    h = jnp.maximum(jnp.dot(h, wb_ref[...], preferred_element_type=jnp.float32)
                    + bb_ref[...], 0.0)
    o_ref[...] = jnp.max(h.reshape(bn, K, -1), axis=1)


def _mlp2_max(m, wa, ba, wb, bb, bn):
    n = m.shape[0] // K
    d = m.shape[1]
    dmid = wa.shape[1]
    dout = wb.shape[1]
    return pl.pallas_call(
        functools.partial(_mlp2_max_body, bn=bn),
        grid=(n // bn,),
        in_specs=[
            pl.BlockSpec((bn * K, d), lambda i: (i, 0)),
            pl.BlockSpec((d, dmid), lambda i: (0, 0)),
            pl.BlockSpec((1, dmid), lambda i: (0, 0)),
            pl.BlockSpec((dmid, dout), lambda i: (0, 0)),
            pl.BlockSpec((1, dout), lambda i: (0, 0)),
        ],
        out_specs=pl.BlockSpec((bn, dout), lambda i: (i, 0)),
        out_shape=jax.ShapeDtypeStruct((n, dout), jnp.float32),
    )(m, wa, ba.reshape(1, dmid), wb, bb.reshape(1, dout))


def _d2_body(q_ref, x1_ref, o_ref):
    q = q_ref[...]
    x1 = x1_ref[...]
    qq = jnp.sum(q * q, axis=1)
    sq = jnp.sum(x1 * x1, axis=1)
    dot = lax.dot_general(q, x1, (((1,), (1,)), ((), ())),
                          preferred_element_type=jnp.float32)
    o_ref[...] = qq[:, None] - 2.0 * dot + sq[None, :]


def _d2(x1, bq):
    return pl.pallas_call(
        _d2_body,
        grid=(N // bq,),
        in_specs=[
            pl.BlockSpec((bq, HID), lambda i: (i, 0)),
            pl.BlockSpec((N, HID), lambda i: (0, 0)),
        ],
        out_specs=pl.BlockSpec((bq, N), lambda i: (i, 0)),
        out_shape=jax.ShapeDtypeStruct((N, N), jnp.float32),
    )(x1, x1)


_RPAD = 10240
_ROWV = _RPAD // L
_SCAP = 2048
_M2 = K * 2 * HID


def _argmin16(bv, biv):
    iota = lax.iota(jnp.int32, L)
    for s in (1, 2, 4, 8):
        ov = _take16(bv, iota ^ s)
        oi = _take16(biv, iota ^ s)
        take = (ov < bv) | ((ov == bv) & (oi < biv))
        bv = jnp.where(take, ov, bv)
        biv = jnp.where(take, oi, biv)
    return bv, biv


def _topk_body(d2_hbm, x1m_hbm, out_hbm,
               rowbuf, svbuf, nbr, nib, m2b, sems):
    wid = lax.axis_index("s") * NC + lax.axis_index("c")
    base = wid * NPW
    nq = jnp.minimum(NPW, N - base)
    iota = lax.iota(jnp.int32, L)
    inf = jnp.full((L,), jnp.inf, jnp.float32)

    for b in range(2):
        def _pad(i, _, b=b):
            rowbuf[pl.ds(b * _RPAD + N + i * L, L)] = inf
            return 0
        lax.fori_loop(0, (_RPAD - N) // L, _pad, 0)
    rowbuf[pl.ds(2 * _RPAD, L)] = inf

    pltpu.async_copy(d2_hbm.at[pl.ds(base * N, N)],
                     rowbuf.at[pl.ds(0, N)], sems.at[0])

    def _row(q, _):
        buf = lax.rem(q, 2)
        bufbase = buf * _RPAD
        pltpu.make_async_copy(d2_hbm.at[pl.ds((base + q) * N, N)],
                              rowbuf.at[pl.ds(bufbase, N)], sems.at[0]).wait()

        @pl.when(q + 1 < nq)
        def _():
            pltpu.async_copy(d2_hbm.at[pl.ds((base + q + 1) * N, N)],
                             rowbuf.at[pl.ds(lax.rem(q + 1, 2) * _RPAD, N)],
                             sems.at[0])

        @pl.when(q >= 2)
        def _():
            pltpu.make_async_copy(
                m2b.at[pl.ds(buf * _M2, _M2)],
                out_hbm.at[pl.ds((base + q - 2) * _M2, _M2)],
                sems.at[3]).wait()

        def _p1(s, accs):
            a = list(accs)
            for u in range(8):
                a[u] = jnp.minimum(a[u],
                                   rowbuf[pl.ds(bufbase + (s * 8 + u) * L, L)])
            return tuple(a)

        accs = lax.fori_loop(0, _ROWV // 8, _p1, (inf,) * 8)
        mv = accs[0]
        for u in range(1, 8):
            mv = jnp.minimum(mv, accs[u])
        thr = mv
        for sh in (1, 2, 4, 8):
            thr = jnp.maximum(thr, _take16(thr, iota ^ sh))

        def _p2(s, cur):
            vs = [rowbuf[pl.ds(bufbase + (s * 8 + u) * L, L)]
                  for u in range(8)]
            mn = vs[0]
            for u in range(1, 8):
                mn = jnp.minimum(mn, vs[u])

            def _ins():
                c = cur
                for u in range(8):
                    m = vs[u] <= thr
                    mi = jnp.where(m, 1, 0)
                    incl = _cumsum16(mi)
                    pos = jnp.where(m, c + incl - mi, _SCAP - L + iota)
                    plsc.store_scatter(svbuf, [pos],
                                       bufbase + (s * 8 + u) * L + iota)
                    c = c + _splat16(incl, 15)
                return c

            return lax.cond(jnp.any(mn <= thr), _ins, lambda: cur)

        curv = lax.fori_loop(0, _ROWV // 8, _p2, jnp.zeros((L,), jnp.int32))
        plsc.store_scatter(svbuf, [curv + iota],
                           jnp.full((L,), 2 * _RPAD, jnp.int32))
        ns = (_scalar16(curv) + L - 1) >> 4

        def _sel(t, res):
            def _scan(g, c):
                bv, biv = c
                iv = svbuf[pl.ds(g * L, L)]
                dv = plsc.load_gather(rowbuf, [iv])
                take = (dv < bv) | ((dv == bv) & (iv < biv))
                return jnp.where(take, dv, bv), jnp.where(take, iv, biv)

            bv, biv = lax.fori_loop(0, ns, _scan,
                                    (inf, jnp.full((L,), 2 * _RPAD, jnp.int32)))
            bv, biv = _argmin16(bv, biv)
            plsc.store_scatter(rowbuf, [biv], inf)
            return jnp.where(iota == t, biv, res)

        res = lax.fori_loop(0, K, _sel, jnp.zeros((L,), jnp.int32))
        nbrs = res - bufbase

        cp1 = pltpu.async_copy(x1m_hbm.at[nbrs], nbr, sems.at[1])
        cp2 = pltpu.async_copy(x1m_hbm.at[pl.ds(base + q, 1)], nib, sems.at[2])
        cp1.wait()
        cp2.wait()
        mb = buf * _M2
        for i in range(K):
            for c in range(HID // L):
                niv = nib[0, pl.ds(c * L, L)]
                njv = nbr[i, pl.ds(c * L, L)]
                m2b[pl.ds(mb + i * 2 * HID + c * L, L)] = niv
                m2b[pl.ds(mb + i * 2 * HID + HID + c * L, L)] = njv - niv
        pltpu.async_copy(m2b.at[pl.ds(mb, _M2)],
                         out_hbm.at[pl.ds((base + q) * _M2, _M2)], sems.at[3])
        return 0

    lax.fori_loop(0, nq, _row, 0)

    for tail in (2, 1):
        pltpu.make_async_copy(
            m2b.at[pl.ds(lax.rem(nq - tail, 2) * _M2, _M2)],
            out_hbm.at[pl.ds((base + nq - tail) * _M2, _M2)],
            sems.at[3]).wait()


def _topk_msg2(d2, x1m):
    mesh = plsc.VectorSubcoreMesh(core_axis_name="c", subcore_axis_name="s")
    kfn = pl.kernel(
        _topk_body,
        out_type=jax.ShapeDtypeStruct((N * K * 2 * HID,), jnp.float32),
        mesh=mesh,
        compiler_params=pltpu.CompilerParams(
            needs_layout_passes=False, use_tc_tiling_on_sc=False),
        scratch_types=[
            pltpu.VMEM((2 * _RPAD + L,), jnp.float32),
            pltpu.VMEM((_SCAP,), jnp.int32),
            pltpu.VMEM((K, HID), jnp.float32),
            pltpu.VMEM((1, HID), jnp.float32),
            pltpu.VMEM((2 * _M2,), jnp.float32),
            pltpu.SemaphoreType.DMA((4,)),
        ],
    )
    return kfn(d2.reshape(N * N), x1m)


def kernel(x, edge_index, W1, b1, W2, b2, W3, b3, W4, b4):
    src = edge_index[0]
    dst = edge_index[1]

    xi = x[dst]
    xj = x[src]
    msg = jnp.concatenate([xi, xj - xi], axis=-1)
    h = _mlp2(msg, W1, b1, W2, b2, block=2560)
    x1 = _segmax(dst, h).reshape(NP, HID)[:N]

    d2 = _d2(x1, bq=400)
    msg2 = _topk_msg2(d2, x1).reshape(N * K, 2 * HID)
    out = _mlp2_max(msg2, W3, b3, W4, b4, bn=80)
    return out


# speedup vs baseline: 3.6752x; 2.0076x over previous
import functools

import jax
import jax.numpy as jnp
from jax import lax
from jax.experimental import pallas as pl
from jax.experimental.pallas import tpu as pltpu
from jax.experimental.pallas import tpu_sc as plsc

N = 10000
E = 640000
K = 16
HID = 64
OUT = 64
IN = 3

NC = 2
NS = 16
NW = NC * NS
L = 16
NPW = 313
NP = NW * NPW


def _take16(x, idx):
    return x.at[idx].get(mode="promise_in_bounds")


def _cumsum16(x):
    iota = lax.iota(jnp.int32, L)
    for s in (1, 2, 4, 8):
        sh = _take16(x, jnp.maximum(iota - s, 0))
        x = x + jnp.where(iota >= s, sh, 0)
    return x


def _splat16(x, i):
    return _take16(x, jnp.full((L,), i, jnp.int32))


def _scalar16(x):
    return jnp.squeeze(lax.slice(x, (0,), (1,)))


_SEG_CH = 8000
_SEG_CAP = 24576
_XROWS = 320


def _segmax_body(dst_hbm, h_hbm, out_hbm, dstbuf, packed, x1l, rows, sems):
    wid = lax.axis_index("s") * NC + lax.axis_index("c")
    base = wid * NPW

    def _z(i, _):
        x1l[pl.ds(i * L, L)] = jnp.zeros((L,), jnp.float32)
        return 0
    lax.fori_loop(0, (_XROWS * HID) // L, _z, 0)

    nchunk = E // _SEG_CH
    cp = pltpu.async_copy(dst_hbm.at[pl.ds(0, _SEG_CH)],
                          dstbuf.at[pl.ds(0, _SEG_CH)], sems.at[0])

    def _chunk(c, cur):
        buf = lax.rem(c, 2)
        nbuf = lax.rem(c + 1, 2)
        pltpu.make_async_copy(dst_hbm.at[pl.ds(c * _SEG_CH, _SEG_CH)],
                              dstbuf.at[pl.ds(buf * _SEG_CH, _SEG_CH)],
                              sems.at[0]).wait()

        @pl.when(c + 1 < nchunk)
        def _():
            pltpu.async_copy(dst_hbm.at[pl.ds((c + 1) * _SEG_CH, _SEG_CH)],
                             dstbuf.at[pl.ds(nbuf * _SEG_CH, _SEG_CH)],
                             sems.at[0])

        def _step(s, cur):
            v = dstbuf[pl.ds(buf * _SEG_CH + s * L, L)]
            l = v - base
            m = (l >= 0) & (l < NPW)
            e = c * _SEG_CH + s * L + lax.iota(jnp.int32, L)
            pk = (e << 9) | jnp.where(m, l, 0)
            mi = jnp.where(m, 1, 0)
            incl = _cumsum16(mi)
            pos = jnp.where(m, cur + incl - mi,
                            _SEG_CAP - L + lax.iota(jnp.int32, L))
            plsc.store_scatter(packed, [pos], pk)
            return cur + _splat16(incl, 15)

        return lax.fori_loop(0, _SEG_CH // L, _step, cur)

    curv = lax.fori_loop(0, nchunk, _chunk, jnp.zeros((L,), jnp.int32))

    plsc.store_scatter(packed, [curv + lax.iota(jnp.int32, L)],
                       jnp.full((L,), 314, jnp.int32))
    cur = _scalar16(curv)
    ngrp = (cur + L - 1) >> 4

    col = lax.iota(jnp.int32, L)

    def _grp(g, _):
        pv = packed[pl.ds(g * L, L)]
        ev = pv >> 9
        lv = pv & 511
        pltpu.async_copy(h_hbm.at[ev], rows, sems.at[1]).wait()
        for i in range(L):
            lsp = _splat16(lv, i)
            for ch in range(HID // L):
                idx = lsp * HID + ch * L + col
                curv = plsc.load_gather(x1l, [idx])
                hv = rows[i, pl.ds(ch * L, L)]
                plsc.store_scatter(x1l, [idx], jnp.maximum(curv, hv))
        return 0

    lax.fori_loop(0, ngrp, _grp, 0)

    pltpu.sync_copy(x1l.at[pl.ds(0, NPW * HID)],
                    out_hbm.at[pl.ds(base * HID, NPW * HID)])


def _segmax(dst, h):
    mesh = plsc.VectorSubcoreMesh(core_axis_name="c", subcore_axis_name="s")
    kfn = pl.kernel(
        _segmax_body,
        out_type=jax.ShapeDtypeStruct((NP * HID,), jnp.float32),
        mesh=mesh,
        compiler_params=pltpu.CompilerParams(
            needs_layout_passes=False, use_tc_tiling_on_sc=False),
        scratch_types=[
            pltpu.VMEM((2 * _SEG_CH,), jnp.int32),
            pltpu.VMEM((_SEG_CAP,), jnp.int32),
            pltpu.VMEM((_XROWS * HID,), jnp.float32),
            pltpu.VMEM((L, HID), jnp.float32),
            pltpu.SemaphoreType.DMA((2,)),
        ],
    )
    return kfn(dst, h)


_EW = E // NW
_ECH = 2000
_MW = 8


def _edgemsg_body(src_hbm, dst_hbm, xt_hbm, out_hbm,
                  xtb, srcb, dstb, msgb, sems):
    wid = lax.axis_index("s") * NC + lax.axis_index("c")
    ebase = wid * _EW
    iota = lax.iota(jnp.int32, L)
    nch = _EW // _ECH

    pltpu.sync_copy(xt_hbm, xtb)

    def _z(i, _):
        msgb[pl.ds(i * L, L)] = jnp.zeros((L,), jnp.float32)
        return 0
    lax.fori_loop(0, (2 * _ECH * _MW) // L, _z, 0)

    pltpu.async_copy(src_hbm.at[pl.ds(ebase, _ECH)],
                     srcb.at[pl.ds(0, _ECH)], sems.at[0])
    pltpu.async_copy(dst_hbm.at[pl.ds(ebase, _ECH)],
                     dstb.at[pl.ds(0, _ECH)], sems.at[1])

    def _chunk(c, _):
        buf = lax.rem(c, 2)
        cb = buf * _ECH
        mb = buf * _ECH * _MW
        pltpu.make_async_copy(src_hbm.at[pl.ds(ebase + c * _ECH, _ECH)],
                              srcb.at[pl.ds(cb, _ECH)], sems.at[0]).wait()
        pltpu.make_async_copy(dst_hbm.at[pl.ds(ebase + c * _ECH, _ECH)],
                              dstb.at[pl.ds(cb, _ECH)], sems.at[1]).wait()

        @pl.when(c + 1 < nch)
        def _():
            nb = lax.rem(c + 1, 2) * _ECH
            pltpu.async_copy(src_hbm.at[pl.ds(ebase + (c + 1) * _ECH, _ECH)],
                             srcb.at[pl.ds(nb, _ECH)], sems.at[0])
            pltpu.async_copy(dst_hbm.at[pl.ds(ebase + (c + 1) * _ECH, _ECH)],
                             dstb.at[pl.ds(nb, _ECH)], sems.at[1])

        @pl.when(c >= 2)
        def _():
            pltpu.make_async_copy(
                msgb.at[pl.ds(mb, _ECH * _MW)],
                out_hbm.at[pl.ds((ebase + (c - 2) * _ECH) * _MW, _ECH * _MW)],
                sems.at[2]).wait()

        def _step(s, _):
            for u in range(5):
                o = (s * 5 + u) * L
                sv = srcb[pl.ds(cb + o, L)]
                dv = dstb[pl.ds(cb + o, L)]
                pos = mb + o * _MW + iota * _MW
                for f in range(IN):
                    xi = plsc.load_gather(xtb, [dv + f * N])
                    xj = plsc.load_gather(xtb, [sv + f * N])
                    plsc.store_scatter(msgb, [pos + f], xi)
                    plsc.store_scatter(msgb, [pos + IN + f], xj - xi)
            return 0

        lax.fori_loop(0, _ECH // (5 * L), _step, 0)
        pltpu.async_copy(msgb.at[pl.ds(mb, _ECH * _MW)],
                         out_hbm.at[pl.ds((ebase + c * _ECH) * _MW,
                                          _ECH * _MW)], sems.at[2])
        return 0

    lax.fori_loop(0, nch, _chunk, 0)
    for tail in (2, 1):
        pltpu.make_async_copy(
            msgb.at[pl.ds(lax.rem(nch - tail, 2) * _ECH * _MW, _ECH * _MW)],
            out_hbm.at[pl.ds((ebase + (nch - tail) * _ECH) * _MW,
                             _ECH * _MW)], sems.at[2]).wait()


def _edgemsg(src, dst, xt):
    mesh = plsc.VectorSubcoreMesh(core_axis_name="c", subcore_axis_name="s")
    kfn = pl.kernel(
        _edgemsg_body,
        out_type=jax.ShapeDtypeStruct((E * _MW,), jnp.float32),
        mesh=mesh,
        compiler_params=pltpu.CompilerParams(
            needs_layout_passes=False, use_tc_tiling_on_sc=False),
        scratch_types=[
            pltpu.VMEM((IN * N,), jnp.float32),
            pltpu.VMEM((2 * _ECH,), jnp.int32),
            pltpu.VMEM((2 * _ECH,), jnp.int32),
            pltpu.VMEM((2 * _ECH * _MW,), jnp.float32),
            pltpu.SemaphoreType.DMA((3,)),
        ],
    )
    return kfn(src, dst, xt)


def _mlp2_body(m_ref, wa_ref, ba_ref, wb_ref, bb_ref, o_ref):
    m = m_ref[...]
    h = jnp.maximum(jnp.dot(m, wa_ref[...], preferred_element_type=jnp.float32)
                    + ba_ref[...], 0.0)
    h = jnp.maximum(jnp.dot(h, wb_ref[...], preferred_element_type=jnp.float32)
                    + bb_ref[...], 0.0)
    o_ref[...] = h


def _mlp2(m, wa, ba, wb, bb, block):
    mm, d = m.shape
    dmid = wa.shape[1]
    dout = wb.shape[1]
    return pl.pallas_call(
        _mlp2_body,
        grid=(mm // block,),
        in_specs=[
            pl.BlockSpec((block, d), lambda i: (i, 0)),
            pl.BlockSpec((d, dmid), lambda i: (0, 0)),
            pl.BlockSpec((1, dmid), lambda i: (0, 0)),
            pl.BlockSpec((dmid, dout), lambda i: (0, 0)),
            pl.BlockSpec((1, dout), lambda i: (0, 0)),
        ],
        out_specs=pl.BlockSpec((block, dout), lambda i: (i, 0)),
        out_shape=jax.ShapeDtypeStruct((mm, dout), jnp.float32),
    )(m, wa, ba.reshape(1, dmid), wb, bb.reshape(1, dout))


def _mlp2_max_body(m_ref, wa_ref, ba_ref, wb_ref, bb_ref, o_ref, *, bn):
    m = m_ref[...]
    h = jnp.maximum(jnp.dot(m, wa_ref[...], preferred_element_type=jnp.float32)
                    + ba_ref[...], 0.0)
    h = jnp.maximum(jnp.dot(h, wb_ref[...], preferred_element_type=jnp.float32)
                    + bb_ref[...], 0.0)
    o_ref[...] = jnp.max(h.reshape(bn, K, -1), axis=1)


def _mlp2_max(m, wa, ba, wb, bb, bn):
    n = m.shape[0] // K
    d = m.shape[1]
    dmid = wa.shape[1]
    dout = wb.shape[1]
    return pl.pallas_call(
        functools.partial(_mlp2_max_body, bn=bn),
        grid=(n // bn,),
        in_specs=[
            pl.BlockSpec((bn * K, d), lambda i: (i, 0)),
            pl.BlockSpec((d, dmid), lambda i: (0, 0)),
            pl.BlockSpec((1, dmid), lambda i: (0, 0)),
            pl.BlockSpec((dmid, dout), lambda i: (0, 0)),
            pl.BlockSpec((1, dout), lambda i: (0, 0)),
        ],
        out_specs=pl.BlockSpec((bn, dout), lambda i: (i, 0)),
        out_shape=jax.ShapeDtypeStruct((n, dout), jnp.float32),
    )(m, wa, ba.reshape(1, dmid), wb, bb.reshape(1, dout))


def _d2_body(q_ref, x1_ref, o_ref):
    q = q_ref[...]
    x1 = x1_ref[...]
    qq = jnp.sum(q * q, axis=1)
    sq = jnp.sum(x1 * x1, axis=1)
    dot = lax.dot_general(q, x1, (((1,), (1,)), ((), ())),
                          preferred_element_type=jnp.float32)
    o_ref[...] = qq[:, None] - 2.0 * dot + sq[None, :]


def _d2(x1, bq):
    return pl.pallas_call(
        _d2_body,
        grid=(N // bq,),
        in_specs=[
            pl.BlockSpec((bq, HID), lambda i: (i, 0)),
            pl.BlockSpec((N, HID), lambda i: (0, 0)),
        ],
        out_specs=pl.BlockSpec((bq, N), lambda i: (i, 0)),
        out_shape=jax.ShapeDtypeStruct((N, N), jnp.float32),
    )(x1, x1)


_RPAD = 10240
_ROWV = _RPAD // L
_SCAP = 2048
_M2 = K * 2 * HID


def _argmin16(bv, biv):
    iota = lax.iota(jnp.int32, L)
    for s in (1, 2, 4, 8):
        ov = _take16(bv, iota ^ s)
        oi = _take16(biv, iota ^ s)
        take = (ov < bv) | ((ov == bv) & (oi < biv))
        bv = jnp.where(take, ov, bv)
        biv = jnp.where(take, oi, biv)
    return bv, biv


def _topk_body(d2_hbm, x1m_hbm, out_hbm,
               rowbuf, svbuf, nbr, nib, m2b, sems):
    wid = lax.axis_index("s") * NC + lax.axis_index("c")
    base = wid * NPW
    nq = jnp.minimum(NPW, N - base)
    iota = lax.iota(jnp.int32, L)
    inf = jnp.full((L,), jnp.inf, jnp.float32)

    for b in range(2):
        def _pad(i, _, b=b):
            rowbuf[pl.ds(b * _RPAD + N + i * L, L)] = inf
            return 0
        lax.fori_loop(0, (_RPAD - N) // L, _pad, 0)
    rowbuf[pl.ds(2 * _RPAD, L)] = inf

    pltpu.async_copy(d2_hbm.at[pl.ds(base * N, N)],
                     rowbuf.at[pl.ds(0, N)], sems.at[0])

    def _row(q, _):
        buf = lax.rem(q, 2)
        bufbase = buf * _RPAD
        pltpu.make_async_copy(d2_hbm.at[pl.ds((base + q) * N, N)],
                              rowbuf.at[pl.ds(bufbase, N)], sems.at[0]).wait()

        @pl.when(q + 1 < nq)
        def _():
            pltpu.async_copy(d2_hbm.at[pl.ds((base + q + 1) * N, N)],
                             rowbuf.at[pl.ds(lax.rem(q + 1, 2) * _RPAD, N)],
                             sems.at[0])

        @pl.when(q >= 2)
        def _():
            pltpu.make_async_copy(
                m2b.at[pl.ds(buf * _M2, _M2)],
                out_hbm.at[pl.ds((base + q - 2) * _M2, _M2)],
                sems.at[3]).wait()

        def _p1(s, accs):
            a = list(accs)
            for u in range(8):
                a[u] = jnp.minimum(a[u],
                                   rowbuf[pl.ds(bufbase + (s * 8 + u) * L, L)])
            return tuple(a)

        accs = lax.fori_loop(0, _ROWV // 8, _p1, (inf,) * 8)
        mv = accs[0]
        for u in range(1, 8):
            mv = jnp.minimum(mv, accs[u])
        thr = mv
        for sh in (1, 2, 4, 8):
            thr = jnp.maximum(thr, _take16(thr, iota ^ sh))

        def _p2(s, cur):
            vs = [rowbuf[pl.ds(bufbase + (s * 8 + u) * L, L)]
                  for u in range(8)]
            mn = vs[0]
            for u in range(1, 8):
                mn = jnp.minimum(mn, vs[u])

            def _ins():
                c = cur
                for u in range(8):
                    m = vs[u] <= thr
                    mi = jnp.where(m, 1, 0)
                    incl = _cumsum16(mi)
                    pos = jnp.where(m, c + incl - mi, _SCAP - L + iota)
                    plsc.store_scatter(svbuf, [pos],
                                       bufbase + (s * 8 + u) * L + iota)
                    c = c + _splat16(incl, 15)
                return c

            return lax.cond(jnp.any(mn <= thr), _ins, lambda: cur)

        curv = lax.fori_loop(0, _ROWV // 8, _p2, jnp.zeros((L,), jnp.int32))
        plsc.store_scatter(svbuf, [curv + iota],
                           jnp.full((L,), 2 * _RPAD, jnp.int32))
        ns = (_scalar16(curv) + L - 1) >> 4

        def _sel(t, res):
            def _scan(g, c):
                bv, biv = c
                iv = svbuf[pl.ds(g * L, L)]
                dv = plsc.load_gather(rowbuf, [iv])
                take = (dv < bv) | ((dv == bv) & (iv < biv))
                return jnp.where(take, dv, bv), jnp.where(take, iv, biv)

            bv, biv = lax.fori_loop(0, ns, _scan,
                                    (inf, jnp.full((L,), 2 * _RPAD, jnp.int32)))
            bv, biv = _argmin16(bv, biv)
            plsc.store_scatter(rowbuf, [biv], inf)
            return jnp.where(iota == t, biv, res)

        res = lax.fori_loop(0, K, _sel, jnp.zeros((L,), jnp.int32))
        nbrs = res - bufbase

        cp1 = pltpu.async_copy(x1m_hbm.at[nbrs], nbr, sems.at[1])
        cp2 = pltpu.async_copy(x1m_hbm.at[pl.ds(base + q, 1)], nib, sems.at[2])
        cp1.wait()
        cp2.wait()
        mb = buf * _M2
        for i in range(K):
            for c in range(HID // L):
                niv = nib[0, pl.ds(c * L, L)]
                njv = nbr[i, pl.ds(c * L, L)]
                m2b[pl.ds(mb + i * 2 * HID + c * L, L)] = niv
                m2b[pl.ds(mb + i * 2 * HID + HID + c * L, L)] = njv - niv
        pltpu.async_copy(m2b.at[pl.ds(mb, _M2)],
                         out_hbm.at[pl.ds((base + q) * _M2, _M2)], sems.at[3])
        return 0

    lax.fori_loop(0, nq, _row, 0)

    for tail in (2, 1):
        pltpu.make_async_copy(
            m2b.at[pl.ds(lax.rem(nq - tail, 2) * _M2, _M2)],
            out_hbm.at[pl.ds((base + nq - tail) * _M2, _M2)],
            sems.at[3]).wait()


def _topk_msg2(d2, x1m):
    mesh = plsc.VectorSubcoreMesh(core_axis_name="c", subcore_axis_name="s")
    kfn = pl.kernel(
        _topk_body,
        out_type=jax.ShapeDtypeStruct((N * K * 2 * HID,), jnp.float32),
        mesh=mesh,
        compiler_params=pltpu.CompilerParams(
            needs_layout_passes=False, use_tc_tiling_on_sc=False),
        scratch_types=[
            pltpu.VMEM((2 * _RPAD + L,), jnp.float32),
            pltpu.VMEM((_SCAP,), jnp.int32),
            pltpu.VMEM((K, HID), jnp.float32),
            pltpu.VMEM((1, HID), jnp.float32),
            pltpu.VMEM((2 * _M2,), jnp.float32),
            pltpu.SemaphoreType.DMA((4,)),
        ],
    )
    return kfn(d2.reshape(N * N), x1m)


def kernel(x, edge_index, W1, b1, W2, b2, W3, b3, W4, b4):
    src = edge_index[0]
    dst = edge_index[1]

    xt = x.T.reshape(IN * N)
    msg = _edgemsg(src, dst, xt).reshape(E, _MW)
    W1p = jnp.concatenate([W1[:IN], W1[IN:], jnp.zeros((_MW - 2 * IN, HID),
                                                       jnp.float32)], axis=0)
    h = _mlp2(msg, W1p, b1, W2, b2, block=2560)
    x1 = _segmax(dst, h).reshape(NP, HID)[:N]

    d2 = _d2(x1, bq=400)
    msg2 = _topk_msg2(d2, x1).reshape(N * K, 2 * HID)
    out = _mlp2_max(msg2, W3, b3, W4, b4, bn=80)
    return out


# speedup vs baseline: 3.9759x; 1.0818x over previous
import functools

import jax
import jax.numpy as jnp
from jax import lax
from jax.experimental import pallas as pl
from jax.experimental.pallas import tpu as pltpu
from jax.experimental.pallas import tpu_sc as plsc

N = 10000
E = 640000
K = 16
HID = 64
OUT = 64
IN = 3

NC = 2
NS = 16
NW = NC * NS
L = 16
NPW = 313
NP = NW * NPW


def _take16(x, idx):
    return x.at[idx].get(mode="promise_in_bounds")


def _cumsum16(x):
    iota = lax.iota(jnp.int32, L)
    for s in (1, 2, 4, 8):
        sh = _take16(x, jnp.maximum(iota - s, 0))
        x = x + jnp.where(iota >= s, sh, 0)
    return x


def _splat16(x, i):
    return _take16(x, jnp.full((L,), i, jnp.int32))


def _scalar16(x):
    return jnp.squeeze(lax.slice(x, (0,), (1,)))


_SEG_CH = 8000
_SEG_CAP = 24576
_XROWS = 320


def _segmax_body(dst_hbm, h_hbm, out_hbm, dstbuf, packed, x1l, rows, sems):
    wid = lax.axis_index("s") * NC + lax.axis_index("c")
    base = wid * NPW
    iota = lax.iota(jnp.int32, L)

    def _z(i, _):
        x1l[pl.ds(i * L, L)] = jnp.zeros((L,), jnp.float32)
        return 0
    lax.fori_loop(0, (_XROWS * HID) // L, _z, 0)

    nchunk = E // _SEG_CH
    pltpu.async_copy(dst_hbm.at[pl.ds(0, _SEG_CH)],
                     dstbuf.at[pl.ds(0, _SEG_CH)], sems.at[0])

    def _chunk(c, cur):
        buf = lax.rem(c, 2)
        pltpu.make_async_copy(dst_hbm.at[pl.ds(c * _SEG_CH, _SEG_CH)],
                              dstbuf.at[pl.ds(buf * _SEG_CH, _SEG_CH)],
                              sems.at[0]).wait()

        @pl.when(c + 1 < nchunk)
        def _():
            pltpu.async_copy(dst_hbm.at[pl.ds((c + 1) * _SEG_CH, _SEG_CH)],
                             dstbuf.at[pl.ds(lax.rem(c + 1, 2) * _SEG_CH,
                                             _SEG_CH)], sems.at[0])

        def _step(s, cur):
            for u in range(5):
                o = (s * 5 + u) * L
                v = dstbuf[pl.ds(buf * _SEG_CH + o, L)]
                l = v - base
                m = (l >= 0) & (l < NPW)
                e = c * _SEG_CH + o + iota
                pk = (e << 9) | jnp.where(m, l, 0)
                mi = jnp.where(m, 1, 0)
                incl = _cumsum16(mi)
                pos = jnp.where(m, cur + incl - mi, _SEG_CAP - L + iota)
                plsc.store_scatter(packed, [pos], pk)
                cur = cur + _splat16(incl, 15)
            return cur

        return lax.fori_loop(0, _SEG_CH // (5 * L), _step, cur)

    curv = lax.fori_loop(0, nchunk, _chunk, jnp.zeros((L,), jnp.int32))

    plsc.store_scatter(packed, [curv + iota], jnp.full((L,), 314, jnp.int32))
    cur = _scalar16(curv)
    ngrp = (cur + L - 1) >> 4

    ev0 = packed[pl.ds(0, L)] >> 9
    pltpu.async_copy(h_hbm.at[ev0], rows.at[0], sems.at[1])

    def _grp(g, _):
        buf = lax.rem(g, 2)
        pltpu.make_async_copy(h_hbm.at[packed[pl.ds(g * L, L)] >> 9],
                              rows.at[buf], sems.at[1]).wait()

        @pl.when(g + 1 < ngrp)
        def _():
            evn = packed[pl.ds((g + 1) * L, L)] >> 9
            pltpu.async_copy(h_hbm.at[evn], rows.at[lax.rem(g + 1, 2)],
                             sems.at[1])

        lv = packed[pl.ds(g * L, L)] & 511
        for i in range(L):
            lsp = _splat16(lv, i)
            for ch in range(HID // L):
                idx = lsp * HID + ch * L + iota
                curx = plsc.load_gather(x1l, [idx])
                hv = rows[buf, i, pl.ds(ch * L, L)]
                plsc.store_scatter(x1l, [idx], jnp.maximum(curx, hv))
        return 0

    lax.fori_loop(0, ngrp, _grp, 0)

    pltpu.sync_copy(x1l.at[pl.ds(0, NPW * HID)],
                    out_hbm.at[pl.ds(base * HID, NPW * HID)])


def _segmax(dst, h):
    mesh = plsc.VectorSubcoreMesh(core_axis_name="c", subcore_axis_name="s")
    kfn = pl.kernel(
        _segmax_body,
        out_type=jax.ShapeDtypeStruct((NP * HID,), jnp.float32),
        mesh=mesh,
        compiler_params=pltpu.CompilerParams(
            needs_layout_passes=False, use_tc_tiling_on_sc=False),
        scratch_types=[
            pltpu.VMEM((2 * _SEG_CH,), jnp.int32),
            pltpu.VMEM((_SEG_CAP,), jnp.int32),
            pltpu.VMEM((_XROWS * HID,), jnp.float32),
            pltpu.VMEM((2, L, HID), jnp.float32),
            pltpu.SemaphoreType.DMA((2,)),
        ],
    )
    return kfn(dst, h)


_EW = E // NW
_ECH = 2000
_MW = 8


def _edgemsg_body(src_hbm, dst_hbm, xt_hbm, out_hbm,
                  xtb, srcb, dstb, msgb, sems):
    wid = lax.axis_index("s") * NC + lax.axis_index("c")
    ebase = wid * _EW
    iota = lax.iota(jnp.int32, L)
    nch = _EW // _ECH

    pltpu.sync_copy(xt_hbm, xtb)

    def _z(i, _):
        msgb[pl.ds(i * L, L)] = jnp.zeros((L,), jnp.float32)
        return 0
    lax.fori_loop(0, (2 * _ECH * _MW) // L, _z, 0)

    pltpu.async_copy(src_hbm.at[pl.ds(ebase, _ECH)],
                     srcb.at[pl.ds(0, _ECH)], sems.at[0])
    pltpu.async_copy(dst_hbm.at[pl.ds(ebase, _ECH)],
                     dstb.at[pl.ds(0, _ECH)], sems.at[1])

    def _chunk(c, _):
        buf = lax.rem(c, 2)
        cb = buf * _ECH
        mb = buf * _ECH * _MW
        pltpu.make_async_copy(src_hbm.at[pl.ds(ebase + c * _ECH, _ECH)],
                              srcb.at[pl.ds(cb, _ECH)], sems.at[0]).wait()
        pltpu.make_async_copy(dst_hbm.at[pl.ds(ebase + c * _ECH, _ECH)],
                              dstb.at[pl.ds(cb, _ECH)], sems.at[1]).wait()

        @pl.when(c + 1 < nch)
        def _():
            nb = lax.rem(c + 1, 2) * _ECH
            pltpu.async_copy(src_hbm.at[pl.ds(ebase + (c + 1) * _ECH, _ECH)],
                             srcb.at[pl.ds(nb, _ECH)], sems.at[0])
            pltpu.async_copy(dst_hbm.at[pl.ds(ebase + (c + 1) * _ECH, _ECH)],
                             dstb.at[pl.ds(nb, _ECH)], sems.at[1])

        @pl.when(c >= 2)
        def _():
            pltpu.make_async_copy(
                msgb.at[pl.ds(mb, _ECH * _MW)],
                out_hbm.at[pl.ds((ebase + (c - 2) * _ECH) * _MW, _ECH * _MW)],
                sems.at[2]).wait()

        def _step(s, _):
            for u in range(5):
                o = (s * 5 + u) * L
                sv = srcb[pl.ds(cb + o, L)]
                dv = dstb[pl.ds(cb + o, L)]
                pos = mb + o * _MW + iota * _MW
                for f in range(IN):
                    xi = plsc.load_gather(xtb, [dv + f * N])
                    xj = plsc.load_gather(xtb, [sv + f * N])
                    plsc.store_scatter(msgb, [pos + f], xi)
                    plsc.store_scatter(msgb, [pos + IN + f], xj - xi)
            return 0

        lax.fori_loop(0, _ECH // (5 * L), _step, 0)
        pltpu.async_copy(msgb.at[pl.ds(mb, _ECH * _MW)],
                         out_hbm.at[pl.ds((ebase + c * _ECH) * _MW,
                                          _ECH * _MW)], sems.at[2])
        return 0

    lax.fori_loop(0, nch, _chunk, 0)
    for tail in (2, 1):
        pltpu.make_async_copy(
            msgb.at[pl.ds(lax.rem(nch - tail, 2) * _ECH * _MW, _ECH * _MW)],
            out_hbm.at[pl.ds((ebase + (nch - tail) * _ECH) * _MW,
                             _ECH * _MW)], sems.at[2]).wait()


def _edgemsg(src, dst, xt):
    mesh = plsc.VectorSubcoreMesh(core_axis_name="c", subcore_axis_name="s")
    kfn = pl.kernel(
        _edgemsg_body,
        out_type=jax.ShapeDtypeStruct((E * _MW,), jnp.float32),
        mesh=mesh,
        compiler_params=pltpu.CompilerParams(
            needs_layout_passes=False, use_tc_tiling_on_sc=False),
        scratch_types=[
            pltpu.VMEM((IN * N,), jnp.float32),
            pltpu.VMEM((2 * _ECH,), jnp.int32),
            pltpu.VMEM((2 * _ECH,), jnp.int32),
            pltpu.VMEM((2 * _ECH * _MW,), jnp.float32),
            pltpu.SemaphoreType.DMA((3,)),
        ],
    )
    return kfn(src, dst, xt)


def _mlp2_body(m_ref, wa_ref, ba_ref, wb_ref, bb_ref, o_ref):
    m = m_ref[...]
    h = jnp.maximum(jnp.dot(m, wa_ref[...], preferred_element_type=jnp.float32)
                    + ba_ref[...], 0.0)
    h = jnp.maximum(jnp.dot(h, wb_ref[...], preferred_element_type=jnp.float32)
                    + bb_ref[...], 0.0)
    o_ref[...] = h


def _mlp2(m, wa, ba, wb, bb, block):
    mm, d = m.shape
    dmid = wa.shape[1]
    dout = wb.shape[1]
    return pl.pallas_call(
        _mlp2_body,
        grid=(mm // block,),
        in_specs=[
            pl.BlockSpec((block, d), lambda i: (i, 0)),
            pl.BlockSpec((d, dmid), lambda i: (0, 0)),
            pl.BlockSpec((1, dmid), lambda i: (0, 0)),
            pl.BlockSpec((dmid, dout), lambda i: (0, 0)),
            pl.BlockSpec((1, dout), lambda i: (0, 0)),
        ],
        out_specs=pl.BlockSpec((block, dout), lambda i: (i, 0)),
        out_shape=jax.ShapeDtypeStruct((mm, dout), jnp.float32),
    )(m, wa, ba.reshape(1, dmid), wb, bb.reshape(1, dout))


def _mlp2_max_body(m_ref, wa_ref, ba_ref, wb_ref, bb_ref, o_ref, *, bn):
    m = m_ref[...]
    h = jnp.maximum(jnp.dot(m, wa_ref[...], preferred_element_type=jnp.float32)
                    + ba_ref[...], 0.0)
    h = jnp.maximum(jnp.dot(h, wb_ref[...], preferred_element_type=jnp.float32)
                    + bb_ref[...], 0.0)
    o_ref[...] = jnp.max(h.reshape(bn, K, -1), axis=1)


def _mlp2_max(m, wa, ba, wb, bb, bn):
    n = m.shape[0] // K
    d = m.shape[1]
    dmid = wa.shape[1]
    dout = wb.shape[1]
    return pl.pallas_call(
        functools.partial(_mlp2_max_body, bn=bn),
        grid=(n // bn,),
        in_specs=[
            pl.BlockSpec((bn * K, d), lambda i: (i, 0)),
            pl.BlockSpec((d, dmid), lambda i: (0, 0)),
            pl.BlockSpec((1, dmid), lambda i: (0, 0)),
            pl.BlockSpec((dmid, dout), lambda i: (0, 0)),
            pl.BlockSpec((1, dout), lambda i: (0, 0)),
        ],
        out_specs=pl.BlockSpec((bn, dout), lambda i: (i, 0)),
        out_shape=jax.ShapeDtypeStruct((n, dout), jnp.float32),
    )(m, wa, ba.reshape(1, dmid), wb, bb.reshape(1, dout))


def _d2_body(q_ref, x1_ref, o_ref):
    q = q_ref[...]
    x1 = x1_ref[...]
    qq = jnp.sum(q * q, axis=1)
    sq = jnp.sum(x1 * x1, axis=1)
    dot = lax.dot_general(q, x1, (((1,), (1,)), ((), ())),
                          preferred_element_type=jnp.float32)
    o_ref[...] = qq[:, None] - 2.0 * dot + sq[None, :]


def _d2(x1, bq):
    return pl.pallas_call(
        _d2_body,
        grid=(N // bq,),
        in_specs=[
            pl.BlockSpec((bq, HID), lambda i: (i, 0)),
            pl.BlockSpec((N, HID), lambda i: (0, 0)),
        ],
        out_specs=pl.BlockSpec((bq, N), lambda i: (i, 0)),
        out_shape=jax.ShapeDtypeStruct((N, N), jnp.float32),
    )(x1, x1)


_RPAD = 10240
_ROWV = _RPAD // L
_SCAP = 2048
_M2 = K * 2 * HID


def _argmin16(bv, biv):
    iota = lax.iota(jnp.int32, L)
    for s in (1, 2, 4, 8):
        ov = _take16(bv, iota ^ s)
        oi = _take16(biv, iota ^ s)
        take = (ov < bv) | ((ov == bv) & (oi < biv))
        bv = jnp.where(take, ov, bv)
        biv = jnp.where(take, oi, biv)
    return bv, biv


def _topk_body(d2_hbm, x1m_hbm, out_hbm,
               rowbuf, svbuf, nbr, nib, m2b, sems):
    wid = lax.axis_index("s") * NC + lax.axis_index("c")
    base = wid * NPW
    nq = jnp.minimum(NPW, N - base)
    iota = lax.iota(jnp.int32, L)
    inf = jnp.full((L,), jnp.inf, jnp.float32)

    for b in range(2):
        def _pad(i, _, b=b):
            rowbuf[pl.ds(b * _RPAD + N + i * L, L)] = inf
            return 0
        lax.fori_loop(0, (_RPAD - N) // L, _pad, 0)
    rowbuf[pl.ds(2 * _RPAD, L)] = inf

    pltpu.async_copy(d2_hbm.at[pl.ds(base * N, N)],
                     rowbuf.at[pl.ds(0, N)], sems.at[0])

    def _row(q, _):
        buf = lax.rem(q, 2)
        bufbase = buf * _RPAD
        pltpu.make_async_copy(d2_hbm.at[pl.ds((base + q) * N, N)],
                              rowbuf.at[pl.ds(bufbase, N)], sems.at[0]).wait()

        @pl.when(q + 1 < nq)
        def _():
            pltpu.async_copy(d2_hbm.at[pl.ds((base + q + 1) * N, N)],
                             rowbuf.at[pl.ds(lax.rem(q + 1, 2) * _RPAD, N)],
                             sems.at[0])

        @pl.when(q >= 2)
        def _():
            pltpu.make_async_copy(
                m2b.at[pl.ds(buf * _M2, _M2)],
                out_hbm.at[pl.ds((base + q - 2) * _M2, _M2)],
                sems.at[3]).wait()

        def _p1(s, accs):
            a = list(accs)
            for u in range(8):
                a[u] = jnp.minimum(a[u],
                                   rowbuf[pl.ds(bufbase + (s * 8 + u) * L, L)])
            return tuple(a)

        accs = lax.fori_loop(0, _ROWV // 8, _p1, (inf,) * 8)
        mv = accs[0]
        for u in range(1, 8):
            mv = jnp.minimum(mv, accs[u])
        thr = mv
        for sh in (1, 2, 4, 8):
            thr = jnp.maximum(thr, _take16(thr, iota ^ sh))

        def _p2(s, cur):
            vs = [rowbuf[pl.ds(bufbase + (s * 8 + u) * L, L)]
                  for u in range(8)]
            mn = vs[0]
            for u in range(1, 8):
                mn = jnp.minimum(mn, vs[u])

            def _ins():
                c = cur
                for u in range(8):
                    m = vs[u] <= thr
                    mi = jnp.where(m, 1, 0)
                    incl = _cumsum16(mi)
                    pos = jnp.where(m, c + incl - mi, _SCAP - L + iota)
                    plsc.store_scatter(svbuf, [pos],
                                       bufbase + (s * 8 + u) * L + iota)
                    c = c + _splat16(incl, 15)
                return c

            return lax.cond(jnp.any(mn <= thr), _ins, lambda: cur)

        curv = lax.fori_loop(0, _ROWV // 8, _p2, jnp.zeros((L,), jnp.int32))
        plsc.store_scatter(svbuf, [curv + iota],
                           jnp.full((L,), 2 * _RPAD, jnp.int32))
        ns = (_scalar16(curv) + L - 1) >> 4

        def _sel(t, res):
            def _scan(g, c):
                bv, biv = c
                iv = svbuf[pl.ds(g * L, L)]
                dv = plsc.load_gather(rowbuf, [iv])
                take = (dv < bv) | ((dv == bv) & (iv < biv))
                return jnp.where(take, dv, bv), jnp.where(take, iv, biv)

            bv, biv = lax.fori_loop(0, ns, _scan,
                                    (inf, jnp.full((L,), 2 * _RPAD, jnp.int32)))
            bv, biv = _argmin16(bv, biv)
            plsc.store_scatter(rowbuf, [biv], inf)
            return jnp.where(iota == t, biv, res)

        res = lax.fori_loop(0, K, _sel, jnp.zeros((L,), jnp.int32))
        nbrs = res - bufbase

        cp1 = pltpu.async_copy(x1m_hbm.at[nbrs], nbr, sems.at[1])
        cp2 = pltpu.async_copy(x1m_hbm.at[pl.ds(base + q, 1)], nib, sems.at[2])
        cp1.wait()
        cp2.wait()
        mb = buf * _M2
        for i in range(K):
            for c in range(HID // L):
                niv = nib[0, pl.ds(c * L, L)]
                njv = nbr[i, pl.ds(c * L, L)]
                m2b[pl.ds(mb + i * 2 * HID + c * L, L)] = niv
                m2b[pl.ds(mb + i * 2 * HID + HID + c * L, L)] = njv - niv
        pltpu.async_copy(m2b.at[pl.ds(mb, _M2)],
                         out_hbm.at[pl.ds((base + q) * _M2, _M2)], sems.at[3])
        return 0

    lax.fori_loop(0, nq, _row, 0)

    for tail in (2, 1):
        pltpu.make_async_copy(
            m2b.at[pl.ds(lax.rem(nq - tail, 2) * _M2, _M2)],
            out_hbm.at[pl.ds((base + nq - tail) * _M2, _M2)],
            sems.at[3]).wait()


def _topk_msg2(d2, x1m):
    mesh = plsc.VectorSubcoreMesh(core_axis_name="c", subcore_axis_name="s")
    kfn = pl.kernel(
        _topk_body,
        out_type=jax.ShapeDtypeStruct((N * K * 2 * HID,), jnp.float32),
        mesh=mesh,
        compiler_params=pltpu.CompilerParams(
            needs_layout_passes=False, use_tc_tiling_on_sc=False),
        scratch_types=[
            pltpu.VMEM((2 * _RPAD + L,), jnp.float32),
            pltpu.VMEM((_SCAP,), jnp.int32),
            pltpu.VMEM((K, HID), jnp.float32),
            pltpu.VMEM((1, HID), jnp.float32),
            pltpu.VMEM((2 * _M2,), jnp.float32),
            pltpu.SemaphoreType.DMA((4,)),
        ],
    )
    return kfn(d2.reshape(N * N), x1m)


def kernel(x, edge_index, W1, b1, W2, b2, W3, b3, W4, b4):
    src = edge_index[0]
    dst = edge_index[1]

    xt = x.T.reshape(IN * N)
    msg = _edgemsg(src, dst, xt).reshape(E, _MW)
    W1p = jnp.concatenate([W1[:IN], W1[IN:], jnp.zeros((_MW - 2 * IN, HID),
                                                       jnp.float32)], axis=0)
    h = _mlp2(msg, W1p, b1, W2, b2, block=2560)
    x1 = _segmax(dst, h).reshape(NP, HID)[:N]

    d2 = _d2(x1, bq=400)
    msg2 = _topk_msg2(d2, x1).reshape(N * K, 2 * HID)
    out = _mlp2_max(msg2, W3, b3, W4, b4, bn=80)
    return out
